# Initial kernel scaffold; baseline (speedup 1.0000x reference)
import jax
import jax.numpy as jnp
from jax.experimental import pallas as pl


def kernel(x, pos_table, use_pos_embed):
    raise NotImplementedError("write your pallas kernel here")


# speedup vs baseline: 1.4925x; 1.4925x over previous
import jax
import jax.numpy as jnp
from jax.experimental import pallas as pl
from jax.experimental.pallas import tpu as pltpu

_S_BLK = 512


def _add_body(scale_ref, x_ref, pos_ref, o_ref):
    o_ref[...] = x_ref[...] + scale_ref[0] * pos_ref[...]


def kernel(x, pos_table, use_pos_embed):
    batch, seq_len, embed_dim = x.shape
    scale = jnp.asarray(use_pos_embed, jnp.float32).reshape((1,))
    grid = (seq_len // _S_BLK, batch)
    out = pl.pallas_call(
        _add_body,
        grid=grid,
        in_specs=[
            pl.BlockSpec(memory_space=pltpu.SMEM),
            pl.BlockSpec((1, _S_BLK, embed_dim), lambda i, b: (b, i, 0)),
            pl.BlockSpec((_S_BLK, embed_dim), lambda i, b: (i, 0)),
        ],
        out_specs=pl.BlockSpec((1, _S_BLK, embed_dim), lambda i, b: (b, i, 0)),
        out_shape=jax.ShapeDtypeStruct(x.shape, x.dtype),
    )(scale, x, pos_table[:seq_len])
    return out


# speedup vs baseline: 1.6569x; 1.1102x over previous
import jax
import jax.numpy as jnp
from jax.experimental import pallas as pl
from jax.experimental.pallas import tpu as pltpu

_S_BLK = 1024


def _add_body(scale_ref, x_ref, pos_ref, o_ref):
    o_ref[...] = x_ref[...] + scale_ref[0] * pos_ref[...]


def kernel(x, pos_table, use_pos_embed):
    batch, seq_len, embed_dim = x.shape
    scale = jnp.asarray(use_pos_embed, jnp.float32).reshape((1,))
    grid = (seq_len // _S_BLK, batch)
    out = pl.pallas_call(
        _add_body,
        grid=grid,
        in_specs=[
            pl.BlockSpec(memory_space=pltpu.SMEM),
            pl.BlockSpec((1, _S_BLK, embed_dim), lambda i, b: (b, i, 0)),
            pl.BlockSpec((_S_BLK, embed_dim), lambda i, b: (i, 0)),
        ],
        out_specs=pl.BlockSpec((1, _S_BLK, embed_dim), lambda i, b: (b, i, 0)),
        out_shape=jax.ShapeDtypeStruct(x.shape, x.dtype),
    )(scale, x, pos_table[:seq_len])
    return out


# speedup vs baseline: 1.7322x; 1.0455x over previous
import jax
import jax.numpy as jnp
from jax.experimental import pallas as pl
from jax.experimental.pallas import tpu as pltpu

_S_BLK = 2048


def _add_body(scale_ref, x_ref, pos_ref, o_ref):
    o_ref[...] = x_ref[...] + scale_ref[0] * pos_ref[...]


def kernel(x, pos_table, use_pos_embed):
    batch, seq_len, embed_dim = x.shape
    scale = jnp.asarray(use_pos_embed, jnp.float32).reshape((1,))
    grid = (seq_len // _S_BLK, batch)
    out = pl.pallas_call(
        _add_body,
        grid=grid,
        in_specs=[
            pl.BlockSpec(memory_space=pltpu.SMEM),
            pl.BlockSpec((1, _S_BLK, embed_dim), lambda i, b: (b, i, 0)),
            pl.BlockSpec((_S_BLK, embed_dim), lambda i, b: (i, 0)),
        ],
        out_specs=pl.BlockSpec((1, _S_BLK, embed_dim), lambda i, b: (b, i, 0)),
        out_shape=jax.ShapeDtypeStruct(x.shape, x.dtype),
    )(scale, x, pos_table[:seq_len])
    return out
